# baseline (speedup 1.0000x reference)
import functools

import jax
import jax.numpy as jnp
import numpy as np
from jax import lax
from jax.experimental import pallas as pl
from jax.experimental.pallas import tpu as pltpu
from jax.experimental.pallas import tpu_sc as plsc

N = 10000
E = 320000
H = 128
NG = 64
COUT = 10
LAYERS = 3

NC = 2
NS = 16
NW = NC * NS
NPAD = 10240
RPW = NPAD // NS
CHUNK = 80
NCHUNKS = 128
EPW = CHUNK * NCHUNKS
EPAD = EPW * NW

RB = 1024
GRID = NPAD // RB

H2 = H // 2

_mesh = plsc.VectorSubcoreMesh(core_axis_name="c", subcore_axis_name="s")


def _pack_bf(y):
    lo = lax.bitcast_convert_type(y[:, :H2], jnp.int32)
    hi = lax.bitcast_convert_type(y[:, H2:], jnp.int32)
    lo_r = lax.shift_right_logical(
        lo + 0x7FFF + (lax.shift_right_logical(lo, 16) & 1), 16
    )
    hi_r = (hi + 0x7FFF + (lax.shift_right_logical(hi, 16) & 1)) & jnp.int32(
        -65536
    )
    return lo_r | hi_r


@functools.partial(
    pl.kernel,
    out_type=jax.ShapeDtypeStruct((NC, NPAD, H), jnp.float32),
    mesh=_mesh,
    scratch_types=[
        pltpu.VMEM((CHUNK,), jnp.int32),
        pltpu.VMEM((CHUNK,), jnp.int32),
        pltpu.VMEM((1, CHUNK), jnp.int32),
        pltpu.VMEM((1, CHUNK), jnp.int32),
        pltpu.VMEM((CHUNK, H2), jnp.int32),
        pltpu.VMEM((CHUNK, H2), jnp.int32),
        pltpu.VMEM((CHUNK, H), jnp.float32),
        pltpu.VMEM_SHARED((NPAD, H), jnp.float32),
        pltpu.SemaphoreType.DMA,
        pltpu.SemaphoreType.DMA,
    ],
    compiler_params=pltpu.CompilerParams(
        needs_layout_passes=False, use_tc_tiling_on_sc=False
    ),
)
def _edge_sum_wide(x_hbm, src_hbm, dst_hbm, out_hbm, s0, s1, d0, d1,
                   braw0, braw1, rowsf, acc, sem0, sem1):
    c = lax.axis_index("c")
    s = lax.axis_index("s")
    wid = s * NC + c
    ebase = wid * EPW
    rbase = s * RPW

    def load_sidx(chunk, buf):
        pltpu.sync_copy(src_hbm.at[pl.ds(ebase + chunk * CHUNK, CHUNK)], buf)

    def load_didx(chunk, buf):
        pltpu.sync_copy(dst_hbm.at[pl.ds(wid * NCHUNKS + chunk, 1)], buf)

    def gather(sbuf, buf, sem):
        return pltpu.async_copy(x_hbm.at[sbuf], buf, sem)

    load_sidx(0, s0)
    load_didx(0, d0)
    gather(s0, braw0, sem0)

    zero = jnp.zeros((16,), jnp.float32)

    def zb(r, carry):
        for j in range(H // 16):
            rowsf[r, pl.ds(j * 16, 16)] = zero
        return carry

    lax.fori_loop(0, CHUNK, zb, 0)
    for j in range(RPW // CHUNK):
        pltpu.sync_copy(rowsf, acc.at[pl.ds(rbase + j * CHUNK, CHUNK)])
    plsc.subcore_barrier()
    load_sidx(1, s1)
    load_didx(1, d1)

    himask = jnp.full((16,), -65536, jnp.int32)

    def widen(braw):
        def crow(i, carry):
            rb = i * 4
            for dr in range(4):
                r = rb + dr
                for g in range(H2 // 16):
                    v = braw[r, pl.ds(g * 16, 16)]
                    rowsf[r, pl.ds(g * 16, 16)] = plsc.bitcast(
                        lax.shift_left(v, 16), jnp.float32
                    )
                    rowsf[r, pl.ds(H2 + g * 16, 16)] = plsc.bitcast(
                        lax.bitwise_and(v, himask), jnp.float32
                    )
            return carry

        lax.fori_loop(0, CHUNK // 4, crow, 0)

    def body(j, carry):
        i0 = 2 * j
        gather(s1, braw1, sem1)
        pltpu.make_async_copy(x_hbm.at[s0], braw0, sem0).wait()
        widen(braw0)
        pltpu.sync_copy(rowsf, acc.at[d0.at[0]], add=True)
        load_sidx(lax.rem(i0 + 2, NCHUNKS), s0)
        load_didx(lax.rem(i0 + 2, NCHUNKS), d0)
        gather(s0, braw0, sem0)
        pltpu.make_async_copy(x_hbm.at[s1], braw1, sem1).wait()
        widen(braw1)
        pltpu.sync_copy(rowsf, acc.at[d1.at[0]], add=True)
        load_sidx(lax.rem(i0 + 3, NCHUNKS), s1)
        load_didx(lax.rem(i0 + 3, NCHUNKS), d1)
        return carry

    lax.fori_loop(0, NCHUNKS // 2, body, 0)
    pltpu.make_async_copy(x_hbm.at[s0], braw0, sem0).wait()
    plsc.subcore_barrier()
    pltpu.sync_copy(acc.at[pl.ds(rbase, RPW)], out_hbm.at[c, pl.ds(rbase, RPW)])


@functools.partial(
    pl.kernel,
    out_type=jax.ShapeDtypeStruct((NC, NPAD), jnp.float32),
    mesh=_mesh,
    scratch_types=[
        pltpu.VMEM((NPAD,), jnp.float32),
        pltpu.VMEM((NPAD,), jnp.float32),
        pltpu.VMEM((EPW,), jnp.int32),
        pltpu.VMEM((EPW,), jnp.int32),
        pltpu.VMEM((RPW,), jnp.float32),
        pltpu.VMEM_SHARED((NS, NPAD), jnp.float32),
    ],
    compiler_params=pltpu.CompilerParams(needs_layout_passes=False),
)
def _edge_sum_scalar(t_hbm, src_hbm, dst_hbm, out_hbm, tv, acc, sb, db, tmp, shacc):
    c = lax.axis_index("c")
    s = lax.axis_index("s")
    wid = s * NC + c

    pltpu.sync_copy(t_hbm, tv)
    zero = jnp.zeros((16,), jnp.float32)

    def z(i, carry):
        acc[pl.ds(i * 16, 16)] = zero
        return carry

    lax.fori_loop(0, NPAD // 16, z, 0)
    pltpu.sync_copy(src_hbm.at[pl.ds(wid * EPW, EPW)], sb)
    pltpu.sync_copy(dst_hbm.at[pl.ds(wid * EPW, EPW)], db)

    def body(i, carry):
        sv = sb[pl.ds(i * 16, 16)]
        dv = db[pl.ds(i * 16, 16)]
        vals = plsc.load_gather(tv, [sv])
        plsc.addupdate_scatter(acc, [dv], vals)
        return carry

    lax.fori_loop(0, EPW // 16, body, 0)

    pltpu.sync_copy(acc, shacc.at[s])
    plsc.subcore_barrier()

    rbase = s * RPW
    pltpu.sync_copy(shacc.at[0, pl.ds(rbase, RPW)], acc.at[pl.ds(0, RPW)])
    for j in range(1, NS):
        pltpu.sync_copy(shacc.at[j, pl.ds(rbase, RPW)], tmp)

        def addk(k, carry):
            acc[pl.ds(k * 16, 16)] = acc[pl.ds(k * 16, 16)] + tmp[pl.ds(k * 16, 16)]
            return carry

        lax.fori_loop(0, RPW // 16, addk, 0)
    pltpu.sync_copy(acc.at[pl.ds(0, RPW)], out_hbm.at[c, pl.ds(rbase, RPW)])


def _lin0_body(x_ref, w_ref, b_ref, o_ref, obf_ref):
    y = (
        jnp.dot(x_ref[...], w_ref[...], preferred_element_type=jnp.float32)
        + b_ref[...]
    )
    o_ref[...] = y
    obf_ref[...] = _pack_bf(y)


def _tc_lin0(x, w, b):
    return pl.pallas_call(
        _lin0_body,
        grid=(GRID,),
        in_specs=[
            pl.BlockSpec((RB, H), lambda i: (i, 0)),
            pl.BlockSpec((H, H), lambda i: (0, 0)),
            pl.BlockSpec((1, H), lambda i: (0, 0)),
        ],
        out_specs=[
            pl.BlockSpec((RB, H), lambda i: (i, 0)),
            pl.BlockSpec((RB, H2), lambda i: (i, 0)),
        ],
        out_shape=[
            jax.ShapeDtypeStruct((NPAD, H), jnp.float32),
            jax.ShapeDtypeStruct((NPAD, H2), jnp.int32),
        ],
    )(x, w, b)


def _tca_body(x_ref, p_ref, deg_ref, a_ref, b_ref, ca_ref, uv_ref, cuv_ref, t_ref):
    invd = 1.0 / jnp.maximum(deg_ref[0] + deg_ref[1], 1.0)
    m = (p_ref[0] + p_ref[1]) * invd[:, None]
    z = (
        jnp.dot(x_ref[...], a_ref[...], preferred_element_type=jnp.float32)
        + jnp.dot(m, b_ref[...], preferred_element_type=jnp.float32)
        + ca_ref[...]
    )
    sig = jax.nn.sigmoid(z)
    t_ref[...] = (
        jnp.dot(sig, uv_ref[...], preferred_element_type=jnp.float32) + cuv_ref[...]
    )


def _tc_a(x, p, degp, a, b, ca, uv, cuv):
    return pl.pallas_call(
        _tca_body,
        grid=(GRID,),
        in_specs=[
            pl.BlockSpec((RB, H), lambda i: (i, 0)),
            pl.BlockSpec((NC, RB, H), lambda i: (0, i, 0)),
            pl.BlockSpec((NC, RB), lambda i: (0, i)),
            pl.BlockSpec((H, H), lambda i: (0, 0)),
            pl.BlockSpec((H, H), lambda i: (0, 0)),
            pl.BlockSpec((1, H), lambda i: (0, 0)),
            pl.BlockSpec((H, H), lambda i: (0, 0)),
            pl.BlockSpec((1, H), lambda i: (0, 0)),
        ],
        out_specs=pl.BlockSpec((RB, H), lambda i: (i, 0)),
        out_shape=jax.ShapeDtypeStruct((NPAD, H), jnp.float32),
    )(x, p, degp, a, b, ca, uv, cuv)


def _tcb_body(x_ref, t1_ref, s2_ref, deg_ref, w_ref, bc_ref, xmbf_ref, xw_ref):
    invd = 1.0 / jnp.maximum(deg_ref[0] + deg_ref[1], 1.0)
    s2 = (s2_ref[0] + s2_ref[1]) * invd
    mv = jax.nn.sigmoid(t1_ref[...] + s2)
    xm = x_ref[...] * mv[:, None]
    xmbf_ref[...] = _pack_bf(xm)
    xw_ref[...] = (
        jnp.dot(xm, w_ref[...], preferred_element_type=jnp.float32) + bc_ref[...]
    )


def _tc_b(x, t1, s2p, degp, w2c, bc):
    return pl.pallas_call(
        _tcb_body,
        grid=(GRID,),
        in_specs=[
            pl.BlockSpec((RB, H), lambda i: (i, 0)),
            pl.BlockSpec((RB,), lambda i: (i,)),
            pl.BlockSpec((NC, RB), lambda i: (0, i)),
            pl.BlockSpec((NC, RB), lambda i: (0, i)),
            pl.BlockSpec((H, H), lambda i: (0, 0)),
            pl.BlockSpec((1, H), lambda i: (0, 0)),
        ],
        out_specs=[
            pl.BlockSpec((RB, H2), lambda i: (i, 0)),
            pl.BlockSpec((RB, H), lambda i: (i, 0)),
        ],
        out_shape=[
            jax.ShapeDtypeStruct((NPAD, H2), jnp.int32),
            jax.ShapeDtypeStruct((NPAD, H), jnp.float32),
        ],
    )(x, t1, s2p, degp, w2c, bc)


def _tcc_body(q_ref, xw_ref, w1_ref, batch_ref, xn_ref, xnbf_ref, pool_ref):
    i = pl.program_id(0)
    aggr = q_ref[0] + q_ref[1]
    xn = jnp.maximum(
        jnp.dot(aggr, w1_ref[...], preferred_element_type=jnp.float32) + xw_ref[...],
        0.0,
    )
    xn_ref[...] = xn
    xnbf_ref[...] = _pack_bf(xn)
    bb = batch_ref[...]
    oh = (
        bb[None, :] == lax.broadcasted_iota(jnp.int32, (NG, RB), 0)
    ).astype(jnp.float32)
    part = jnp.dot(oh, xn, preferred_element_type=jnp.float32)

    @pl.when(i == 0)
    def _():
        pool_ref[...] = part

    @pl.when(i > 0)
    def _():
        pool_ref[...] += part


def _tc_c(q, xw, w1c, batch_pad):
    return pl.pallas_call(
        _tcc_body,
        grid=(GRID,),
        in_specs=[
            pl.BlockSpec((NC, RB, H), lambda i: (0, i, 0)),
            pl.BlockSpec((RB, H), lambda i: (i, 0)),
            pl.BlockSpec((H, H), lambda i: (0, 0)),
            pl.BlockSpec((RB,), lambda i: (i,)),
        ],
        out_specs=[
            pl.BlockSpec((RB, H), lambda i: (i, 0)),
            pl.BlockSpec((RB, H2), lambda i: (i, 0)),
            pl.BlockSpec((NG, H), lambda i: (0, 0)),
        ],
        out_shape=[
            jax.ShapeDtypeStruct((NPAD, H), jnp.float32),
            jax.ShapeDtypeStruct((NPAD, H2), jnp.int32),
            jax.ShapeDtypeStruct((NG, H), jnp.float32),
        ],
    )(q, xw, w1c, batch_pad)


def _tcf_body(x1_ref, x2_ref, x3_ref, w1_ref, b1_ref, w2_ref, b2_ref, o_ref):
    h = jnp.concatenate([x1_ref[...], x2_ref[...], x3_ref[...]], axis=1)
    h = jnp.maximum(
        jnp.dot(h, w1_ref[...], preferred_element_type=jnp.float32) + b1_ref[...],
        0.0,
    )
    h = jnp.dot(h, w2_ref[...], preferred_element_type=jnp.float32) + b2_ref[...]
    mx = jnp.max(h, axis=1, keepdims=True)
    z = h - mx
    o_ref[...] = z - jnp.log(jnp.sum(jnp.exp(z), axis=1, keepdims=True))


def _tc_final(x1, x2, x3, w1, b1, w2, b2):
    return pl.pallas_call(
        _tcf_body,
        out_shape=jax.ShapeDtypeStruct((NG, COUT), jnp.float32),
    )(x1, x2, x3, w1, b1, w2, b2)


def kernel(x, params, edge_index, batch):
    f32 = jnp.float32
    src = edge_index[0].astype(jnp.int32)
    dst = edge_index[1].astype(jnp.int32)
    pad_e = EPAD - E
    src_p = jnp.concatenate([src, jnp.zeros((pad_e,), jnp.int32)])
    dst_p = jnp.concatenate(
        [dst, N + (jnp.arange(pad_e, dtype=jnp.int32) % (NPAD - N))]
    )
    dst2d = dst_p.reshape(EPAD // CHUNK, CHUNK)
    x_pad = jnp.concatenate([x, jnp.zeros((NPAD - N, x.shape[1]), f32)])
    batch_pad = jnp.concatenate(
        [batch.astype(jnp.int32), jnp.full((NPAD - N,), NG, jnp.int32)]
    )
    ones = jnp.ones((NPAD,), f32)

    folds = []
    for i in range(LAYERS):
        p1 = params["ma1h"][i]
        p2 = params["ma2h"][i]
        w3 = p1["lin3"]["w"]
        a = p1["lin2"]["w"] @ w3[:H]
        b = p1["lin1"]["w"] @ w3[H:]
        ca = (p1["lin2"]["b"] @ w3[:H] + p1["lin1"]["b"] @ w3[H:] + p1["lin3"]["b"])
        w3p = p2["lin3"]["w"]
        u = p2["lin2"]["w"] @ w3p[:H]
        v = p2["lin1"]["w"] @ w3p[H:]
        c2 = (
            p2["lin2"]["b"] @ w3p[:H] + p2["lin1"]["b"] @ w3p[H:] + p2["lin3"]["b"]
        )[0]
        uv = jnp.concatenate([u, v, jnp.zeros((H, H - 2), f32)], axis=1)
        cuv = jnp.zeros((1, H), f32).at[0, 0].set(c2)
        pc = params["conv"][i]
        bc = (pc["lin1"]["b"] + pc["lin2"]["b"])[None, :]
        folds.append(
            dict(a=a, b=b, ca=ca[None, :], uv=uv, cuv=cuv,
                 w1c=pc["lin1"]["w"], w2c=pc["lin2"]["w"], bc=bc)
        )

    degp = _edge_sum_scalar(ones, src_p, dst_p)
    xc, xbf = _tc_lin0(x_pad, params["lin0"]["w"], params["lin0"]["b"][None, :])

    pools = []
    for i in range(LAYERS):
        f = folds[i]
        p = _edge_sum_wide(xbf, src_p, dst2d)
        t = _tc_a(xc, p, degp, f["a"], f["b"], f["ca"], f["uv"], f["cuv"])
        t1 = t[:, 0]
        t2 = t[:, 1]
        s2p = _edge_sum_scalar(t2, src_p, dst_p)
        xmbf, xw = _tc_b(xc, t1, s2p, degp, f["w2c"], f["bc"])
        q = _edge_sum_wide(xmbf, src_p, dst2d)
        xc, xbf, pool_i = _tc_c(q, xw, f["w1c"], batch_pad)
        pools.append(pool_i)

    return _tc_final(
        pools[0], pools[1], pools[2],
        params["lin1"]["w"], params["lin1"]["b"][None, :],
        params["lin2"]["w"], params["lin2"]["b"][None, :],
    )


# speedup vs baseline: 1.1733x; 1.1733x over previous
import functools

import jax
import jax.numpy as jnp
import numpy as np
from jax import lax
from jax.experimental import pallas as pl
from jax.experimental.pallas import tpu as pltpu
from jax.experimental.pallas import tpu_sc as plsc

N = 10000
E = 320000
H = 128
NG = 64
COUT = 10
LAYERS = 3

NC = 2
NS = 16
NW = NC * NS
NPAD = 10240
RPW = NPAD // NS
CHUNK = 80
NCHUNKS = 128
EPW = CHUNK * NCHUNKS
EPAD = EPW * NW

RB = 1024
GRID = NPAD // RB

H2 = H // 2

_mesh = plsc.VectorSubcoreMesh(core_axis_name="c", subcore_axis_name="s")


def _pack_bf(y):
    lo = lax.bitcast_convert_type(y[:, :H2], jnp.int32)
    hi = lax.bitcast_convert_type(y[:, H2:], jnp.int32)
    lo_r = lax.shift_right_logical(
        lo + 0x7FFF + (lax.shift_right_logical(lo, 16) & 1), 16
    )
    hi_r = (hi + 0x7FFF + (lax.shift_right_logical(hi, 16) & 1)) & jnp.int32(
        -65536
    )
    return lo_r | hi_r


@functools.partial(
    pl.kernel,
    out_type=jax.ShapeDtypeStruct((NC, 2, NPAD, H2), jnp.float32),
    mesh=_mesh,
    scratch_types=[
        pltpu.VMEM((CHUNK,), jnp.int32),
        pltpu.VMEM((CHUNK,), jnp.int32),
        pltpu.VMEM((1, CHUNK), jnp.int32),
        pltpu.VMEM((1, CHUNK), jnp.int32),
        pltpu.VMEM((CHUNK, H2), jnp.int32),
        pltpu.VMEM((CHUNK, H2), jnp.int32),
        pltpu.VMEM((CHUNK, H2), jnp.float32),
        pltpu.VMEM((CHUNK, H2), jnp.float32),
        pltpu.VMEM((CHUNK, H2), jnp.float32),
        pltpu.VMEM((CHUNK, H2), jnp.float32),
        pltpu.VMEM_SHARED((NPAD, H2), jnp.float32),
        pltpu.VMEM_SHARED((NPAD, H2), jnp.float32),
        pltpu.SemaphoreType.DMA,
        pltpu.SemaphoreType.DMA,
        pltpu.SemaphoreType.DMA,
        pltpu.SemaphoreType.DMA,
    ],
    compiler_params=pltpu.CompilerParams(
        needs_layout_passes=False, use_tc_tiling_on_sc=False
    ),
)
def _edge_sum_wide(x_hbm, src_hbm, dst_hbm, out_hbm, s0, s1, d0, d1,
                   braw0, braw1, rl0, rr0, rl1, rr1, accL, accR,
                   semG0, semG1, semS0, semS1):
    c = lax.axis_index("c")
    s = lax.axis_index("s")
    wid = s * NC + c
    ebase = wid * EPW
    rbase = s * RPW

    def load_sidx(chunk, buf):
        pltpu.sync_copy(src_hbm.at[pl.ds(ebase + chunk * CHUNK, CHUNK)], buf)

    def load_didx(chunk, buf):
        pltpu.sync_copy(dst_hbm.at[pl.ds(wid * NCHUNKS + chunk, 1)], buf)

    def gather(sbuf, buf, sem):
        return pltpu.async_copy(x_hbm.at[sbuf], buf, sem)

    load_sidx(0, s0)
    gather(s0, braw0, semG0)
    load_sidx(1, s1)
    gather(s1, braw1, semG1)

    zero = jnp.zeros((16,), jnp.float32)

    def zb(r, carry):
        for j in range(H2 // 16):
            rl0[r, pl.ds(j * 16, 16)] = zero
        return carry

    lax.fori_loop(0, CHUNK, zb, 0)
    for j in range(RPW // CHUNK):
        pltpu.sync_copy(rl0, accL.at[pl.ds(rbase + j * CHUNK, CHUNK)])
        pltpu.sync_copy(rl0, accR.at[pl.ds(rbase + j * CHUNK, CHUNK)])
    plsc.subcore_barrier()

    himask = jnp.full((16,), -65536, jnp.int32)

    def widen(braw, rl, rr):
        def crow(i, carry):
            rb = i * 4
            for dr in range(4):
                r = rb + dr
                for g in range(H2 // 16):
                    v = braw[r, pl.ds(g * 16, 16)]
                    rl[r, pl.ds(g * 16, 16)] = plsc.bitcast(
                        lax.shift_left(v, 16), jnp.float32
                    )
                    rr[r, pl.ds(g * 16, 16)] = plsc.bitcast(
                        lax.bitwise_and(v, himask), jnp.float32
                    )
            return carry

        lax.fori_loop(0, CHUNK // 4, crow, 0)

    def half(j, i0, sbuf, dbuf, braw, rl, rr, semG, semS):
        pltpu.make_async_copy(x_hbm.at[sbuf], braw, semG).wait()

        @pl.when(j > 0)
        def _():
            pltpu.make_async_copy(rl, accL.at[dbuf.at[0]], semS).wait()
            pltpu.make_async_copy(rr, accR.at[dbuf.at[0]], semS).wait()

        load_didx(i0, dbuf)
        load_sidx(lax.rem(i0 + 2, NCHUNKS), sbuf)
        widen(braw, rl, rr)
        gather(sbuf, braw, semG)
        pltpu.async_copy(rl, accL.at[dbuf.at[0]], semS, add=True)
        pltpu.async_copy(rr, accR.at[dbuf.at[0]], semS, add=True)

    def body(j, carry):
        i0 = 2 * j
        half(j, i0, s0, d0, braw0, rl0, rr0, semG0, semS0)
        half(j, i0 + 1, s1, d1, braw1, rl1, rr1, semG1, semS1)
        return carry

    lax.fori_loop(0, NCHUNKS // 2, body, 0)
    pltpu.make_async_copy(x_hbm.at[s0], braw0, semG0).wait()
    pltpu.make_async_copy(x_hbm.at[s1], braw1, semG1).wait()
    pltpu.make_async_copy(rl0, accL.at[d0.at[0]], semS0).wait()
    pltpu.make_async_copy(rr0, accR.at[d0.at[0]], semS0).wait()
    pltpu.make_async_copy(rl1, accL.at[d1.at[0]], semS1).wait()
    pltpu.make_async_copy(rr1, accR.at[d1.at[0]], semS1).wait()
    plsc.subcore_barrier()
    pltpu.sync_copy(accL.at[pl.ds(rbase, RPW)],
                    out_hbm.at[c, 0, pl.ds(rbase, RPW)])
    pltpu.sync_copy(accR.at[pl.ds(rbase, RPW)],
                    out_hbm.at[c, 1, pl.ds(rbase, RPW)])


@functools.partial(
    pl.kernel,
    out_type=jax.ShapeDtypeStruct((NC, NPAD), jnp.float32),
    mesh=_mesh,
    scratch_types=[
        pltpu.VMEM((NPAD,), jnp.float32),
        pltpu.VMEM((NPAD,), jnp.float32),
        pltpu.VMEM((EPW,), jnp.int32),
        pltpu.VMEM((EPW,), jnp.int32),
        pltpu.VMEM((RPW,), jnp.float32),
        pltpu.VMEM_SHARED((NS, NPAD), jnp.float32),
    ],
    compiler_params=pltpu.CompilerParams(needs_layout_passes=False),
)
def _edge_sum_scalar(t_hbm, src_hbm, dst_hbm, out_hbm, tv, acc, sb, db, tmp, shacc):
    c = lax.axis_index("c")
    s = lax.axis_index("s")
    wid = s * NC + c

    pltpu.sync_copy(t_hbm, tv)
    zero = jnp.zeros((16,), jnp.float32)

    def z(i, carry):
        acc[pl.ds(i * 16, 16)] = zero
        return carry

    lax.fori_loop(0, NPAD // 16, z, 0)
    pltpu.sync_copy(src_hbm.at[pl.ds(wid * EPW, EPW)], sb)
    pltpu.sync_copy(dst_hbm.at[pl.ds(wid * EPW, EPW)], db)

    def body(i, carry):
        sv = sb[pl.ds(i * 16, 16)]
        dv = db[pl.ds(i * 16, 16)]
        vals = plsc.load_gather(tv, [sv])
        plsc.addupdate_scatter(acc, [dv], vals)
        return carry

    lax.fori_loop(0, EPW // 16, body, 0)

    pltpu.sync_copy(acc, shacc.at[s])
    plsc.subcore_barrier()

    rbase = s * RPW
    pltpu.sync_copy(shacc.at[0, pl.ds(rbase, RPW)], acc.at[pl.ds(0, RPW)])
    for j in range(1, NS):
        pltpu.sync_copy(shacc.at[j, pl.ds(rbase, RPW)], tmp)

        def addk(k, carry):
            acc[pl.ds(k * 16, 16)] = acc[pl.ds(k * 16, 16)] + tmp[pl.ds(k * 16, 16)]
            return carry

        lax.fori_loop(0, RPW // 16, addk, 0)
    pltpu.sync_copy(acc.at[pl.ds(0, RPW)], out_hbm.at[c, pl.ds(rbase, RPW)])


def _lin0_body(x_ref, w_ref, b_ref, o_ref, obf_ref):
    y = (
        jnp.dot(x_ref[...], w_ref[...], preferred_element_type=jnp.float32)
        + b_ref[...]
    )
    o_ref[...] = y
    obf_ref[...] = _pack_bf(y)


def _tc_lin0(x, w, b):
    return pl.pallas_call(
        _lin0_body,
        grid=(GRID,),
        in_specs=[
            pl.BlockSpec((RB, H), lambda i: (i, 0)),
            pl.BlockSpec((H, H), lambda i: (0, 0)),
            pl.BlockSpec((1, H), lambda i: (0, 0)),
        ],
        out_specs=[
            pl.BlockSpec((RB, H), lambda i: (i, 0)),
            pl.BlockSpec((RB, H2), lambda i: (i, 0)),
        ],
        out_shape=[
            jax.ShapeDtypeStruct((NPAD, H), jnp.float32),
            jax.ShapeDtypeStruct((NPAD, H2), jnp.int32),
        ],
    )(x, w, b)


def _tca_body(x_ref, p_ref, deg_ref, a_ref, b_ref, ca_ref, uv_ref, cuv_ref, t_ref):
    invd = 1.0 / jnp.maximum(deg_ref[0] + deg_ref[1], 1.0)
    m = jnp.concatenate(
        [p_ref[0, 0] + p_ref[1, 0], p_ref[0, 1] + p_ref[1, 1]], axis=1
    ) * invd[:, None]
    z = (
        jnp.dot(x_ref[...], a_ref[...], preferred_element_type=jnp.float32)
        + jnp.dot(m, b_ref[...], preferred_element_type=jnp.float32)
        + ca_ref[...]
    )
    sig = jax.nn.sigmoid(z)
    t_ref[...] = (
        jnp.dot(sig, uv_ref[...], preferred_element_type=jnp.float32) + cuv_ref[...]
    )


def _tc_a(x, p, degp, a, b, ca, uv, cuv):
    return pl.pallas_call(
        _tca_body,
        grid=(GRID,),
        in_specs=[
            pl.BlockSpec((RB, H), lambda i: (i, 0)),
            pl.BlockSpec((NC, 2, RB, H2), lambda i: (0, 0, i, 0)),
            pl.BlockSpec((NC, RB), lambda i: (0, i)),
            pl.BlockSpec((H, H), lambda i: (0, 0)),
            pl.BlockSpec((H, H), lambda i: (0, 0)),
            pl.BlockSpec((1, H), lambda i: (0, 0)),
            pl.BlockSpec((H, H), lambda i: (0, 0)),
            pl.BlockSpec((1, H), lambda i: (0, 0)),
        ],
        out_specs=pl.BlockSpec((RB, H), lambda i: (i, 0)),
        out_shape=jax.ShapeDtypeStruct((NPAD, H), jnp.float32),
    )(x, p, degp, a, b, ca, uv, cuv)


def _tcb_body(x_ref, t1_ref, s2_ref, deg_ref, w_ref, bc_ref, xmbf_ref, xw_ref):
    invd = 1.0 / jnp.maximum(deg_ref[0] + deg_ref[1], 1.0)
    s2 = (s2_ref[0] + s2_ref[1]) * invd
    mv = jax.nn.sigmoid(t1_ref[...] + s2)
    xm = x_ref[...] * mv[:, None]
    xmbf_ref[...] = _pack_bf(xm)
    xw_ref[...] = (
        jnp.dot(xm, w_ref[...], preferred_element_type=jnp.float32) + bc_ref[...]
    )


def _tc_b(x, t1, s2p, degp, w2c, bc):
    return pl.pallas_call(
        _tcb_body,
        grid=(GRID,),
        in_specs=[
            pl.BlockSpec((RB, H), lambda i: (i, 0)),
            pl.BlockSpec((RB,), lambda i: (i,)),
            pl.BlockSpec((NC, RB), lambda i: (0, i)),
            pl.BlockSpec((NC, RB), lambda i: (0, i)),
            pl.BlockSpec((H, H), lambda i: (0, 0)),
            pl.BlockSpec((1, H), lambda i: (0, 0)),
        ],
        out_specs=[
            pl.BlockSpec((RB, H2), lambda i: (i, 0)),
            pl.BlockSpec((RB, H), lambda i: (i, 0)),
        ],
        out_shape=[
            jax.ShapeDtypeStruct((NPAD, H2), jnp.int32),
            jax.ShapeDtypeStruct((NPAD, H), jnp.float32),
        ],
    )(x, t1, s2p, degp, w2c, bc)


def _tcc_body(q_ref, xw_ref, w1_ref, batch_ref, xn_ref, xnbf_ref, pool_ref):
    i = pl.program_id(0)
    aggr = jnp.concatenate(
        [q_ref[0, 0] + q_ref[1, 0], q_ref[0, 1] + q_ref[1, 1]], axis=1
    )
    xn = jnp.maximum(
        jnp.dot(aggr, w1_ref[...], preferred_element_type=jnp.float32) + xw_ref[...],
        0.0,
    )
    xn_ref[...] = xn
    xnbf_ref[...] = _pack_bf(xn)
    bb = batch_ref[...]
    oh = (
        bb[None, :] == lax.broadcasted_iota(jnp.int32, (NG, RB), 0)
    ).astype(jnp.float32)
    part = jnp.dot(oh, xn, preferred_element_type=jnp.float32)

    @pl.when(i == 0)
    def _():
        pool_ref[...] = part

    @pl.when(i > 0)
    def _():
        pool_ref[...] += part


def _tc_c(q, xw, w1c, batch_pad):
    return pl.pallas_call(
        _tcc_body,
        grid=(GRID,),
        in_specs=[
            pl.BlockSpec((NC, 2, RB, H2), lambda i: (0, 0, i, 0)),
            pl.BlockSpec((RB, H), lambda i: (i, 0)),
            pl.BlockSpec((H, H), lambda i: (0, 0)),
            pl.BlockSpec((RB,), lambda i: (i,)),
        ],
        out_specs=[
            pl.BlockSpec((RB, H), lambda i: (i, 0)),
            pl.BlockSpec((RB, H2), lambda i: (i, 0)),
            pl.BlockSpec((NG, H), lambda i: (0, 0)),
        ],
        out_shape=[
            jax.ShapeDtypeStruct((NPAD, H), jnp.float32),
            jax.ShapeDtypeStruct((NPAD, H2), jnp.int32),
            jax.ShapeDtypeStruct((NG, H), jnp.float32),
        ],
    )(q, xw, w1c, batch_pad)


def _tcf_body(x1_ref, x2_ref, x3_ref, w1_ref, b1_ref, w2_ref, b2_ref, o_ref):
    h = jnp.concatenate([x1_ref[...], x2_ref[...], x3_ref[...]], axis=1)
    h = jnp.maximum(
        jnp.dot(h, w1_ref[...], preferred_element_type=jnp.float32) + b1_ref[...],
        0.0,
    )
    h = jnp.dot(h, w2_ref[...], preferred_element_type=jnp.float32) + b2_ref[...]
    mx = jnp.max(h, axis=1, keepdims=True)
    z = h - mx
    o_ref[...] = z - jnp.log(jnp.sum(jnp.exp(z), axis=1, keepdims=True))


def _tc_final(x1, x2, x3, w1, b1, w2, b2):
    return pl.pallas_call(
        _tcf_body,
        out_shape=jax.ShapeDtypeStruct((NG, COUT), jnp.float32),
    )(x1, x2, x3, w1, b1, w2, b2)


def kernel(x, params, edge_index, batch):
    f32 = jnp.float32
    src = edge_index[0].astype(jnp.int32)
    dst = edge_index[1].astype(jnp.int32)
    pad_e = EPAD - E
    src_p = jnp.concatenate([src, jnp.zeros((pad_e,), jnp.int32)])
    dst_p = jnp.concatenate(
        [dst, N + (jnp.arange(pad_e, dtype=jnp.int32) % (NPAD - N))]
    )
    dst2d = dst_p.reshape(EPAD // CHUNK, CHUNK)
    x_pad = jnp.concatenate([x, jnp.zeros((NPAD - N, x.shape[1]), f32)])
    batch_pad = jnp.concatenate(
        [batch.astype(jnp.int32), jnp.full((NPAD - N,), NG, jnp.int32)]
    )
    ones = jnp.ones((NPAD,), f32)

    folds = []
    for i in range(LAYERS):
        p1 = params["ma1h"][i]
        p2 = params["ma2h"][i]
        w3 = p1["lin3"]["w"]
        a = p1["lin2"]["w"] @ w3[:H]
        b = p1["lin1"]["w"] @ w3[H:]
        ca = (p1["lin2"]["b"] @ w3[:H] + p1["lin1"]["b"] @ w3[H:] + p1["lin3"]["b"])
        w3p = p2["lin3"]["w"]
        u = p2["lin2"]["w"] @ w3p[:H]
        v = p2["lin1"]["w"] @ w3p[H:]
        c2 = (
            p2["lin2"]["b"] @ w3p[:H] + p2["lin1"]["b"] @ w3p[H:] + p2["lin3"]["b"]
        )[0]
        uv = jnp.concatenate([u, v, jnp.zeros((H, H - 2), f32)], axis=1)
        cuv = jnp.zeros((1, H), f32).at[0, 0].set(c2)
        pc = params["conv"][i]
        bc = (pc["lin1"]["b"] + pc["lin2"]["b"])[None, :]
        folds.append(
            dict(a=a, b=b, ca=ca[None, :], uv=uv, cuv=cuv,
                 w1c=pc["lin1"]["w"], w2c=pc["lin2"]["w"], bc=bc)
        )

    degp = _edge_sum_scalar(ones, src_p, dst_p)
    xc, xbf = _tc_lin0(x_pad, params["lin0"]["w"], params["lin0"]["b"][None, :])

    pools = []
    for i in range(LAYERS):
        f = folds[i]
        p = _edge_sum_wide(xbf, src_p, dst2d)
        t = _tc_a(xc, p, degp, f["a"], f["b"], f["ca"], f["uv"], f["cuv"])
        t1 = t[:, 0]
        t2 = t[:, 1]
        s2p = _edge_sum_scalar(t2, src_p, dst_p)
        xmbf, xw = _tc_b(xc, t1, s2p, degp, f["w2c"], f["bc"])
        q = _edge_sum_wide(xmbf, src_p, dst2d)
        xc, xbf, pool_i = _tc_c(q, xw, f["w1c"], batch_pad)
        pools.append(pool_i)

    return _tc_final(
        pools[0], pools[1], pools[2],
        params["lin1"]["w"], params["lin1"]["b"][None, :],
        params["lin2"]["w"], params["lin2"]["b"][None, :],
    )
